# Initial kernel scaffold; baseline (speedup 1.0000x reference)
import jax
import jax.numpy as jnp
from jax.experimental import pallas as pl


def kernel(x, edge_index, edge_attr, global_attr, Wl0, Wr0, We0, Wg0, att0, Wl1, Wr1, We1, Wg1, att1, Wl2, We2, Wg2, att2):
    raise NotImplementedError("write your pallas kernel here")


# speedup vs baseline: 7.2807x; 7.2807x over previous
import jax
import jax.numpy as jnp
from jax import lax
from jax.experimental import pallas as pl
from jax.experimental.pallas import tpu as pltpu
from jax.experimental.pallas import tpu_sc as plsc

F32 = jnp.float32
N_REAL = 10000
N_PAD = 10240
E_TOT = 320000
BLK_N = 512
BLK_E = 512
C_EDGE = 80
N_TILES = 32
E_PER_TILE = E_TOT // N_TILES
ROWS_PER_TILE = N_PAD // 16


def _leaky(v):
    return jnp.maximum(v, 0.2 * v)


def _hsum16(v):
    dnums = lax.GatherDimensionNumbers(
        offset_dims=(), collapsed_slice_dims=(0,), start_index_map=(0,))
    for k in (8, 4, 2, 1):
        idx = lax.iota(jnp.int32, 16) ^ k
        v = v + lax.gather(v, idx[:, None], dnums, slice_sizes=(1,),
                           mode=lax.GatherScatterMode.PROMISE_IN_BOUNDS)
    return v


def _proj_from_h(h, wl_ref, wr_ref, g_ref, wg_ref, xl_ref, xr_ref):
    xl_ref[...] = jnp.dot(h, wl_ref[...], preferred_element_type=F32)
    gp = jnp.dot(g_ref[...], wg_ref[...], preferred_element_type=F32)[0:1, :]
    xr_ref[...] = jnp.dot(h, wr_ref[...], preferred_element_type=F32) + gp


def _prep0_body(x_ref, wl_ref, wr_ref, g_ref, wg_ref, xl_ref, xr_ref):
    _proj_from_h(x_ref[...], wl_ref, wr_ref, g_ref, wg_ref, xl_ref, xr_ref)


def _hdiv(part_ref, den_ref):
    s = part_ref[0] + part_ref[1]
    dsum = jnp.sum(den_ref[...], axis=1, keepdims=True)
    return s / jnp.maximum(dsum, 1e-30)


def _prep1_body(part_ref, den_ref, wl_ref, wr_ref, g_ref, wg_ref, xl_ref, xr_ref):
    _proj_from_h(_hdiv(part_ref, den_ref), wl_ref, wr_ref, g_ref, wg_ref,
                 xl_ref, xr_ref)


def _prep2_body(part_ref, den_ref, wl_ref, g_ref, wg_ref, xl_ref, xr_ref):
    h = _hdiv(part_ref, den_ref)
    xl = jnp.dot(h, wl_ref[...], preferred_element_type=F32)
    xl_ref[...] = xl
    gp = jnp.dot(g_ref[...], wg_ref[...], preferred_element_type=F32)[0:1, :]
    xr_ref[...] = xl + gp


def _eproj_body(ea_ref, we0_ref, we1_ref, we2_ref, e0_ref, e1_ref, e2_ref):
    ea = ea_ref[...]
    e0_ref[...] = jnp.dot(ea, we0_ref[...], preferred_element_type=F32)
    e1_ref[...] = jnp.dot(ea, we1_ref[...], preferred_element_type=F32)
    e2_ref[...] = jnp.dot(ea, we2_ref[...], preferred_element_type=F32)


def _final_body(part_ref, den_ref, gum_ref, row_ref, col_ref, lp_ref):
    s = part_ref[0] + part_ref[1]
    dsum = jnp.sum(den_ref[...], axis=1, keepdims=True)
    logits = s / jnp.maximum(dsum, 1e-30)
    rows_i = lax.broadcasted_iota(jnp.int32, (N_PAD, 16), 0)
    cols_i = lax.broadcasted_iota(jnp.int32, (N_PAD, 16), 1)
    valid = (rows_i < N_REAL) & (cols_i < 8)
    lmask = jnp.where(valid, logits, -1e30)
    mx = jnp.max(lmask)
    se = jnp.sum(jnp.where(valid, jnp.exp(lmask - mx), 0.0))
    lse = mx + jnp.log(se)
    score = jnp.where(valid, lmask + gum_ref[...], -1e30)
    ms = jnp.max(score)
    fidx = rows_i * 8 + cols_i
    am = jnp.min(jnp.where(score >= ms, fidx, jnp.int32(2 ** 30)))
    row_ref[0, 0] = am // 8
    col_ref[0, 0] = am % 8
    sel = jnp.sum(jnp.where((fidx == am) & valid, lmask, 0.0))
    lp_ref[0, 0] = sel - lse


def _rep(shape):
    return pl.BlockSpec(shape, lambda i: tuple(0 for _ in shape))


def _nblk(width):
    return pl.BlockSpec((BLK_N, width), lambda i: (i, 0))


def _tc_prep0(xp, Wl, Wr, g8, Wg):
    return pl.pallas_call(
        _prep0_body,
        grid=(N_PAD // BLK_N,),
        in_specs=[_nblk(128), _rep((128, 128)), _rep((128, 128)),
                  _rep((8, 16)), _rep((16, 128))],
        out_specs=[_nblk(128), _nblk(128)],
        out_shape=[jax.ShapeDtypeStruct((N_PAD, 128), F32),
                   jax.ShapeDtypeStruct((N_PAD, 128), F32)],
    )(xp, Wl, Wr, g8, Wg)


def _tc_prep1(part, den_t, Wl, Wr, g8, Wg):
    return pl.pallas_call(
        _prep1_body,
        grid=(N_PAD // BLK_N,),
        in_specs=[pl.BlockSpec((2, BLK_N, 128), lambda i: (0, i, 0)),
                  pl.BlockSpec((BLK_N, 2), lambda i: (i, 0)),
                  _rep((128, 128)), _rep((128, 128)),
                  _rep((8, 16)), _rep((16, 128))],
        out_specs=[_nblk(128), _nblk(128)],
        out_shape=[jax.ShapeDtypeStruct((N_PAD, 128), F32),
                   jax.ShapeDtypeStruct((N_PAD, 128), F32)],
    )(part, den_t, Wl, Wr, g8, Wg)


def _tc_prep2(part, den_t, Wlp, g8, Wgp):
    return pl.pallas_call(
        _prep2_body,
        grid=(N_PAD // BLK_N,),
        in_specs=[pl.BlockSpec((2, BLK_N, 128), lambda i: (0, i, 0)),
                  pl.BlockSpec((BLK_N, 2), lambda i: (i, 0)),
                  _rep((128, 16)), _rep((8, 16)), _rep((16, 16))],
        out_specs=[_nblk(16), _nblk(16)],
        out_shape=[jax.ShapeDtypeStruct((N_PAD, 16), F32),
                   jax.ShapeDtypeStruct((N_PAD, 16), F32)],
    )(part, den_t, Wlp, g8, Wgp)


def _tc_eproj(ea, We0, We1, We2p):
    return pl.pallas_call(
        _eproj_body,
        grid=(E_TOT // BLK_E,),
        in_specs=[pl.BlockSpec((BLK_E, 16), lambda i: (i, 0)),
                  _rep((16, 128)), _rep((16, 128)), _rep((16, 16))],
        out_specs=[pl.BlockSpec((BLK_E, 128), lambda i: (i, 0)),
                   pl.BlockSpec((BLK_E, 128), lambda i: (i, 0)),
                   pl.BlockSpec((BLK_E, 16), lambda i: (i, 0))],
        out_shape=[jax.ShapeDtypeStruct((E_TOT, 128), F32),
                   jax.ShapeDtypeStruct((E_TOT, 128), F32),
                   jax.ShapeDtypeStruct((E_TOT, 16), F32)],
    )(ea, We0, We1, We2p)


def _tc_final(part2, den2_t, gum2d):
    return pl.pallas_call(
        _final_body,
        in_specs=[pl.BlockSpec(memory_space=pltpu.VMEM),
                  pl.BlockSpec(memory_space=pltpu.VMEM),
                  pl.BlockSpec(memory_space=pltpu.VMEM)],
        out_specs=[pl.BlockSpec(memory_space=pltpu.SMEM),
                   pl.BlockSpec(memory_space=pltpu.SMEM),
                   pl.BlockSpec(memory_space=pltpu.SMEM)],
        out_shape=[jax.ShapeDtypeStruct((1, 1), jnp.int32),
                   jax.ShapeDtypeStruct((1, 1), jnp.int32),
                   jax.ShapeDtypeStruct((1, 1), F32)],
    )(part2, den2_t, gum2d)


def _make_sc_pass(de):
    jg = de // 16
    n_chunks = E_PER_TILE // C_EDGE
    n_groups = C_EDGE // 16
    mesh = plsc.VectorSubcoreMesh(core_axis_name="c", subcore_axis_name="s")

    def body(xl_hbm, xr_hbm, ep_hbm, src_hbm, dst_hbm, att_hbm,
             num_out, den_out,
             src_v, dst_v, a_v, b_v, e_v, w_v, att_v, p_v, zden_v,
             num_sh, den_sh, sem_a, sem_b):
        cid = lax.axis_index("c")
        sid = lax.axis_index("s")
        zero = jnp.zeros((16,), F32)
        lane = lax.iota(jnp.int32, 16)

        def zrow(i, _):
            for j in range(jg):
                w_v[i, pl.ds(j * 16, 16)] = zero
            return 0
        lax.fori_loop(0, C_EDGE, zrow, 0)

        def zden(i, _):
            zden_v[pl.ds(i * 16, 16)] = zero
            return 0
        lax.fori_loop(0, ROWS_PER_TILE // 16, zden, 0)

        nbase = sid * ROWS_PER_TILE
        for q in range(ROWS_PER_TILE // C_EDGE):
            pltpu.sync_copy(w_v, num_sh.at[pl.ds(nbase + q * C_EDGE, C_EDGE)])
        pltpu.sync_copy(zden_v, den_sh.at[pl.ds(nbase, ROWS_PER_TILE)])
        plsc.subcore_barrier()

        pltpu.sync_copy(att_hbm, att_v)
        atts = [att_v[pl.ds(j * 16, 16)] for j in range(jg)]
        ebase = (cid * 16 + sid) * E_PER_TILE

        def chunk(t, _):
            base = pl.multiple_of(ebase + t * C_EDGE, 8)
            pltpu.sync_copy(src_hbm.at[pl.ds(base, C_EDGE)], src_v)
            pltpu.sync_copy(dst_hbm.at[pl.ds(base, C_EDGE)], dst_v)
            cp_a = pltpu.async_copy(xl_hbm.at[src_v], a_v, sem_a)
            cp_b = pltpu.async_copy(xr_hbm.at[dst_v], b_v, sem_b)
            pltpu.sync_copy(ep_hbm.at[pl.ds(base, C_EDGE)], e_v)
            cp_a.wait()
            cp_b.wait()

            def group(g, _):
                p16 = zero
                for e in range(16):
                    i = g * 16 + e
                    avs = [a_v[i, pl.ds(j * 16, 16)] for j in range(jg)]
                    acc = zero
                    for j in range(jg):
                        m = avs[j] + b_v[i, pl.ds(j * 16, 16)] \
                            + e_v[i, pl.ds(j * 16, 16)]
                        acc = acc + _leaky(m) * atts[j]
                    p = jnp.exp(_hsum16(acc))
                    for j in range(jg):
                        w_v[i, pl.ds(j * 16, 16)] = avs[j] * p
                    p16 = jnp.where(lane == e, p, p16)
                p_v[pl.ds(g * 16, 16)] = p16
                return 0
            lax.fori_loop(0, n_groups, group, 0)
            pltpu.sync_copy(w_v, num_sh.at[dst_v], add=True)
            pltpu.sync_copy(p_v, den_sh.at[dst_v], add=True)
            return 0
        lax.fori_loop(0, n_chunks, chunk, 0)
        plsc.subcore_barrier()
        pltpu.sync_copy(num_sh.at[pl.ds(nbase, ROWS_PER_TILE)],
                        num_out.at[cid, pl.ds(nbase, ROWS_PER_TILE)])
        pltpu.sync_copy(den_sh.at[pl.ds(nbase, ROWS_PER_TILE)],
                        den_out.at[cid, pl.ds(nbase, ROWS_PER_TILE)])

    return pl.kernel(
        body,
        out_type=[jax.ShapeDtypeStruct((2, N_PAD, de), F32),
                  jax.ShapeDtypeStruct((2, N_PAD), F32)],
        mesh=mesh,
        compiler_params=pltpu.CompilerParams(use_tc_tiling_on_sc=False),
        scratch_types=[
            pltpu.VMEM((C_EDGE,), jnp.int32),
            pltpu.VMEM((C_EDGE,), jnp.int32),
            pltpu.VMEM((C_EDGE, de), F32),
            pltpu.VMEM((C_EDGE, de), F32),
            pltpu.VMEM((C_EDGE, de), F32),
            pltpu.VMEM((C_EDGE, de), F32),
            pltpu.VMEM((de,), F32),
            pltpu.VMEM((C_EDGE,), F32),
            pltpu.VMEM((ROWS_PER_TILE,), F32),
            pltpu.VMEM_SHARED((N_PAD, de), F32),
            pltpu.VMEM_SHARED((N_PAD,), F32),
            pltpu.SemaphoreType.DMA,
            pltpu.SemaphoreType.DMA,
        ],
    )


_sc_pass_wide = _make_sc_pass(128)
_sc_pass_narrow = _make_sc_pass(16)


def kernel(x, edge_index, edge_attr, global_attr, Wl0, Wr0, We0, Wg0, att0,
           Wl1, Wr1, We1, Wg1, att1, Wl2, We2, Wg2, att2):
    src = edge_index[0]
    dst = edge_index[1]
    xp = jnp.pad(x, ((0, N_PAD - N_REAL), (0, 0)))
    g8 = jnp.pad(global_attr[None, :], ((0, 7), (0, 0)))
    We2p = jnp.pad(We2, ((0, 0), (0, 8)))
    Wl2p = jnp.pad(Wl2, ((0, 0), (0, 8)))
    Wg2p = jnp.pad(Wg2, ((0, 0), (0, 8)))
    att2p = jnp.pad(att2, (0, 8))

    u = jax.random.uniform(jax.random.key(42), (N_REAL * 8,))
    gum = -jnp.log(-jnp.log(u + 1e-20) + 1e-20)
    gum2d = jnp.pad(gum.reshape(N_REAL, 8), ((0, N_PAD - N_REAL), (0, 8)))

    ep0, ep1, ep2 = _tc_eproj(edge_attr, We0, We1, We2p)

    xl0, xr0 = _tc_prep0(xp, Wl0, Wr0, g8, Wg0)
    num0, den0 = _sc_pass_wide(xl0, xr0, ep0, src, dst, att0)

    xl1, xr1 = _tc_prep1(num0, den0.T, Wl1, Wr1, g8, Wg1)
    num1, den1 = _sc_pass_wide(xl1, xr1, ep1, src, dst, att1)

    xl2, xr2 = _tc_prep2(num1, den1.T, Wl2p, g8, Wg2p)
    num2, den2 = _sc_pass_narrow(xl2, xr2, ep2, src, dst, att2p)

    row2, col2, lp2 = _tc_final(num2, den2.T, gum2d)
    return row2[0, 0], col2[0, 0], lp2[0, 0]
